# baseline (speedup 1.0000x reference)
import functools

import jax
import jax.numpy as jnp
from jax.experimental import pallas as pl
from jax.experimental.pallas import tpu as pltpu
from jax.experimental.pallas import tpu_sc as plsc

B, S, V, H = 64, 512, 30522, 768
N_TOK = B * S
TWOD = 1024
LN_EPS = 1e-12

NC, NS = 2, 16
NW = NC * NS
NCHUNK = 4
CB = N_TOK // NCHUNK
B_PER_W = CB // NW
CH = 64
NCH = B_PER_W // CH
TB = 512


def _sc_word_gather(weight, ids1d):
    @functools.partial(
        pl.kernel,
        out_type=jax.ShapeDtypeStruct((CB, H), weight.dtype),
        mesh=plsc.VectorSubcoreMesh(
            core_axis_name="c", subcore_axis_name="s", num_cores=NC,
            num_subcores=NS),
        scratch_types=[
            pltpu.VMEM((B_PER_W,), jnp.int32),
            pltpu.VMEM((CH, H), jnp.float32),
            pltpu.SemaphoreType.DMA,
        ],
    )
    def gather_kernel(w_hbm, i_hbm, o_hbm, idx_v, rows_v, sem):
        wid = jax.lax.axis_index("s") * NC + jax.lax.axis_index("c")
        base = wid * B_PER_W
        pltpu.sync_copy(i_hbm.at[pl.ds(base, B_PER_W)], idx_v)

        @pl.loop(0, NCH)
        def _(i):
            pltpu.async_copy(
                w_hbm.at[idx_v.at[pl.ds(i * CH, CH)]], rows_v, sem).wait()
            pltpu.sync_copy(rows_v, o_hbm.at[pl.ds(base + i * CH, CH)])

    return gather_kernel(weight, ids1d)


def _tc_body(bbox_ref, g_ref, x_ref, y_ref, h_ref, w_ref, p_ref, t_ref,
             gam_ref, bet_ref, o_ref):
    bb = bbox_ref[...]
    left = bb[:, 0:1]
    upper = bb[:, 1:2]
    right = bb[:, 2:3]
    lower = bb[:, 3:4]
    hh = lower - upper
    ww = right - left
    iota = jax.lax.broadcasted_iota(jnp.int32, (TB, TWOD), 1)

    def onehot(idx):
        return (iota == idx).astype(jnp.bfloat16)

    cx = onehot(left) + onehot(right)
    cy = onehot(upper) + onehot(lower)
    ch = onehot(hh)
    cw = onehot(ww)

    acc = g_ref[...] + p_ref[...] + t_ref[0:1, :]
    acc = acc + jnp.dot(cx, x_ref[...], preferred_element_type=jnp.float32)
    acc = acc + jnp.dot(cy, y_ref[...], preferred_element_type=jnp.float32)
    acc = acc + jnp.dot(ch, h_ref[...], preferred_element_type=jnp.float32)
    acc = acc + jnp.dot(cw, w_ref[...], preferred_element_type=jnp.float32)

    mean = jnp.mean(acc, axis=1, keepdims=True)
    cen = acc - mean
    var = jnp.mean(cen * cen, axis=1, keepdims=True)
    o_ref[...] = (cen * jax.lax.rsqrt(var + LN_EPS)) * gam_ref[...] + bet_ref[...]


def _tc_combine(bbox2, gathered, xb, yb, hb, wb, pe, tte, gam2, bet2):
    return pl.pallas_call(
        _tc_body,
        grid=(CB // TB,),
        in_specs=[
            pl.BlockSpec((TB, 4), lambda i: (i, 0)),
            pl.BlockSpec((TB, H), lambda i: (i, 0)),
            pl.BlockSpec((TWOD, H), lambda i: (0, 0)),
            pl.BlockSpec((TWOD, H), lambda i: (0, 0)),
            pl.BlockSpec((TWOD, H), lambda i: (0, 0)),
            pl.BlockSpec((TWOD, H), lambda i: (0, 0)),
            pl.BlockSpec((S, H), lambda i: (0, 0)),
            pl.BlockSpec((2, H), lambda i: (0, 0)),
            pl.BlockSpec((1, H), lambda i: (0, 0)),
            pl.BlockSpec((1, H), lambda i: (0, 0)),
        ],
        out_specs=pl.BlockSpec((TB, H), lambda i: (i, 0)),
        out_shape=jax.ShapeDtypeStruct((CB, H), jnp.float32),
        compiler_params=pltpu.CompilerParams(
            dimension_semantics=("arbitrary",)),
    )(bbox2, gathered, xb, yb, hb, wb, pe, tte, gam2, bet2)


def kernel(input_ids, bbox, weight, token_type_embeddings,
           position_embeddings, x_position_embeddings, y_position_embeddings,
           h_position_embeddings, w_position_embeddings, ln_gamma, ln_beta):
    ids1d = input_ids.reshape(N_TOK)
    bbox2 = bbox.reshape(N_TOK, 4)
    xb = x_position_embeddings.astype(jnp.bfloat16)
    yb = y_position_embeddings.astype(jnp.bfloat16)
    hb = h_position_embeddings.astype(jnp.bfloat16)
    wb = w_position_embeddings.astype(jnp.bfloat16)
    gam2 = ln_gamma.reshape(1, H)
    bet2 = ln_beta.reshape(1, H)

    outs = []
    for c in range(NCHUNK):
        gathered = _sc_word_gather(weight, ids1d[c * CB:(c + 1) * CB])
        outs.append(_tc_combine(
            bbox2[c * CB:(c + 1) * CB], gathered, xb, yb, hb, wb,
            position_embeddings, token_type_embeddings, gam2, bet2))
    return jnp.concatenate(outs, axis=0).reshape(B, S, H)


# speedup vs baseline: 1.0009x; 1.0009x over previous
import functools

import jax
import jax.numpy as jnp
from jax.experimental import pallas as pl
from jax.experimental.pallas import tpu as pltpu
from jax.experimental.pallas import tpu_sc as plsc

B, S, V, H = 64, 512, 30522, 768
N_TOK = B * S
TWOD = 1024
LN_EPS = 1e-12

NC, NS = 2, 16
NW = NC * NS
NCHUNK = 4
CB = N_TOK // NCHUNK
B_PER_W = CB // NW
CH = 64
NCH = B_PER_W // CH
TB = 512


def _sc_word_gather(weight, ids1d):
    @functools.partial(
        pl.kernel,
        out_type=jax.ShapeDtypeStruct((CB, H), weight.dtype),
        mesh=plsc.VectorSubcoreMesh(
            core_axis_name="c", subcore_axis_name="s", num_cores=NC,
            num_subcores=NS),
        scratch_types=[
            pltpu.VMEM((B_PER_W,), jnp.int32),
            pltpu.VMEM((CH, H), jnp.float32),
            pltpu.SemaphoreType.DMA,
        ],
    )
    def gather_kernel(w_hbm, i_hbm, o_hbm, idx_v, rows_v, sem):
        wid = jax.lax.axis_index("s") * NC + jax.lax.axis_index("c")
        base = wid * B_PER_W
        pltpu.sync_copy(i_hbm.at[pl.ds(base, B_PER_W)], idx_v)

        @pl.loop(0, NCH)
        def _(i):
            pltpu.async_copy(
                w_hbm.at[idx_v.at[pl.ds(i * CH, CH)]], rows_v, sem).wait()
            pltpu.sync_copy(rows_v, o_hbm.at[pl.ds(base + i * CH, CH)])

    return gather_kernel(weight, ids1d)


def _tc_body(bbox_ref, g_ref, x_ref, y_ref, h_ref, w_ref, p_ref, t_ref,
             gam_ref, bet_ref, o_ref):
    bb = bbox_ref[...]
    left = bb[:, 0:1]
    upper = bb[:, 1:2]
    right = bb[:, 2:3]
    lower = bb[:, 3:4]
    hh = lower - upper
    ww = right - left
    iota = jax.lax.broadcasted_iota(jnp.int32, (TB, TWOD), 1)

    def onehot(idx):
        return (iota == idx).astype(jnp.bfloat16)

    cx = onehot(left) + onehot(right)
    cy = onehot(upper) + onehot(lower)
    ch = onehot(hh)
    cw = onehot(ww)

    acc = g_ref[...] + p_ref[...] + t_ref[0:1, :]
    acc = acc + jnp.dot(cx, x_ref[...], preferred_element_type=jnp.float32)
    acc = acc + jnp.dot(cy, y_ref[...], preferred_element_type=jnp.float32)
    acc = acc + jnp.dot(ch, h_ref[...], preferred_element_type=jnp.float32)
    acc = acc + jnp.dot(cw, w_ref[...], preferred_element_type=jnp.float32)

    mean = jnp.mean(acc, axis=1, keepdims=True)
    cen = acc - mean
    var = jnp.mean(cen * cen, axis=1, keepdims=True)
    o_ref[...] = (cen * jax.lax.rsqrt(var + LN_EPS)) * gam_ref[...] + bet_ref[...]


def _tc_combine(bbox2, gathered, xb, yb, hb, wb, pe, tte, gam2, bet2):
    return pl.pallas_call(
        _tc_body,
        grid=(CB // TB,),
        in_specs=[
            pl.BlockSpec((TB, 4), lambda i: (i, 0)),
            pl.BlockSpec((TB, H), lambda i: (i, 0)),
            pl.BlockSpec((TWOD, H), lambda i: (0, 0)),
            pl.BlockSpec((TWOD, H), lambda i: (0, 0)),
            pl.BlockSpec((TWOD, H), lambda i: (0, 0)),
            pl.BlockSpec((TWOD, H), lambda i: (0, 0)),
            pl.BlockSpec((S, H), lambda i: (0, 0)),
            pl.BlockSpec((2, H), lambda i: (0, 0)),
            pl.BlockSpec((1, H), lambda i: (0, 0)),
            pl.BlockSpec((1, H), lambda i: (0, 0)),
        ],
        out_specs=pl.BlockSpec((TB, H), lambda i: (i, 0)),
        out_shape=jax.ShapeDtypeStruct((CB, H), jnp.float32),
        compiler_params=pltpu.CompilerParams(
            dimension_semantics=("arbitrary",)),
    )(bbox2, gathered, xb, yb, hb, wb, pe, tte, gam2, bet2)


def kernel(input_ids, bbox, weight, token_type_embeddings,
           position_embeddings, x_position_embeddings, y_position_embeddings,
           h_position_embeddings, w_position_embeddings, ln_gamma, ln_beta):
    ids1d = input_ids.reshape(N_TOK)
    bbox2 = bbox.reshape(N_TOK, 4)
    xb = x_position_embeddings.astype(jnp.bfloat16)
    yb = y_position_embeddings.astype(jnp.bfloat16)
    hb = h_position_embeddings.astype(jnp.bfloat16)
    wb = w_position_embeddings.astype(jnp.bfloat16)
    gam2 = ln_gamma.reshape(1, H)
    bet2 = ln_beta.reshape(1, H)

    gs = [_sc_word_gather(weight, ids1d[c * CB:(c + 1) * CB])
          for c in range(NCHUNK)]
    outs = []
    for c in range(NCHUNK):
        outs.append(_tc_combine(
            bbox2[c * CB:(c + 1) * CB], gs[c], xb, yb, hb, wb,
            position_embeddings, token_type_embeddings, gam2, bet2))
    return jnp.concatenate(outs, axis=0).reshape(B, S, H)


# speedup vs baseline: 1.2540x; 1.2529x over previous
import functools

import jax
import jax.numpy as jnp
from jax.experimental import pallas as pl
from jax.experimental.pallas import tpu as pltpu
from jax.experimental.pallas import tpu_sc as plsc

B, S, V, H = 64, 512, 30522, 768
N_TOK = B * S
TWOD = 1024
LN_EPS = 1e-12

NC, NS = 2, 16
NW = NC * NS
NCHUNK = 1
CB = N_TOK // NCHUNK
B_PER_W = CB // NW
CH = 64
NCH = B_PER_W // CH
TB = 512


def _sc_word_gather(weight, ids1d):
    @functools.partial(
        pl.kernel,
        out_type=jax.ShapeDtypeStruct((CB, H), weight.dtype),
        mesh=plsc.VectorSubcoreMesh(
            core_axis_name="c", subcore_axis_name="s", num_cores=NC,
            num_subcores=NS),
        scratch_types=[
            pltpu.VMEM((B_PER_W,), jnp.int32),
            pltpu.VMEM((CH, H), jnp.float32),
            pltpu.SemaphoreType.DMA,
        ],
    )
    def gather_kernel(w_hbm, i_hbm, o_hbm, idx_v, rows_v, sem):
        wid = jax.lax.axis_index("s") * NC + jax.lax.axis_index("c")
        base = wid * B_PER_W
        pltpu.sync_copy(i_hbm.at[pl.ds(base, B_PER_W)], idx_v)

        @pl.loop(0, NCH)
        def _(i):
            pltpu.async_copy(
                w_hbm.at[idx_v.at[pl.ds(i * CH, CH)]], rows_v, sem).wait()
            pltpu.sync_copy(rows_v, o_hbm.at[pl.ds(base + i * CH, CH)])

    return gather_kernel(weight, ids1d)


NSPLIT = 2
SB = TB // NSPLIT


def _tc_body(bbox_ref, g_ref, tab_ref, p_ref, t_ref, gam_ref, bet_ref, o_ref):
    for k in range(NSPLIT):
        sl = pl.ds(k * SB, SB)
        bb = bbox_ref[sl, :]
        left = bb[:, 0:1]
        upper = bb[:, 1:2]
        right = bb[:, 2:3]
        lower = bb[:, 3:4]
        iota = jax.lax.broadcasted_iota(jnp.int32, (SB, TWOD), 1)

        def onehot(idx):
            return (iota == idx).astype(jnp.bfloat16)

        cc = jnp.concatenate([
            onehot(left) + onehot(right),
            onehot(upper) + onehot(lower),
            onehot(lower - upper),
            onehot(right - left),
        ], axis=1)

        acc = g_ref[sl, :] + p_ref[sl, :] + t_ref[0:1, :]
        acc = acc + jnp.dot(cc, tab_ref[...],
                            preferred_element_type=jnp.float32)

        mean = jnp.mean(acc, axis=1, keepdims=True)
        cen = acc - mean
        var = jnp.mean(cen * cen, axis=1, keepdims=True)
        o_ref[sl, :] = ((cen * jax.lax.rsqrt(var + LN_EPS)) * gam_ref[...]
                        + bet_ref[...])


def _tc_combine(bbox2, gathered, tab4, pe, tte, gam2, bet2):
    return pl.pallas_call(
        _tc_body,
        grid=(CB // TB,),
        in_specs=[
            pl.BlockSpec((TB, 4), lambda i: (i, 0)),
            pl.BlockSpec((TB, H), lambda i: (i, 0)),
            pl.BlockSpec((4 * TWOD, H), lambda i: (0, 0)),
            pl.BlockSpec((S, H), lambda i: (0, 0)),
            pl.BlockSpec((2, H), lambda i: (0, 0)),
            pl.BlockSpec((1, H), lambda i: (0, 0)),
            pl.BlockSpec((1, H), lambda i: (0, 0)),
        ],
        out_specs=pl.BlockSpec((TB, H), lambda i: (i, 0)),
        out_shape=jax.ShapeDtypeStruct((CB, H), jnp.float32),
        compiler_params=pltpu.CompilerParams(
            dimension_semantics=("arbitrary",)),
    )(bbox2, gathered, tab4, pe, tte, gam2, bet2)


def kernel(input_ids, bbox, weight, token_type_embeddings,
           position_embeddings, x_position_embeddings, y_position_embeddings,
           h_position_embeddings, w_position_embeddings, ln_gamma, ln_beta):
    ids1d = input_ids.reshape(N_TOK)
    bbox2 = bbox.reshape(N_TOK, 4)
    tab4 = jnp.concatenate([
        x_position_embeddings, y_position_embeddings,
        h_position_embeddings, w_position_embeddings,
    ], axis=0).astype(jnp.bfloat16)
    gam2 = ln_gamma.reshape(1, H)
    bet2 = ln_beta.reshape(1, H)

    gs = [_sc_word_gather(weight, ids1d[c * CB:(c + 1) * CB])
          for c in range(NCHUNK)]
    outs = []
    for c in range(NCHUNK):
        outs.append(_tc_combine(
            bbox2[c * CB:(c + 1) * CB], gs[c], tab4,
            position_embeddings, token_type_embeddings, gam2, bet2))
    return jnp.concatenate(outs, axis=0).reshape(B, S, H)


# speedup vs baseline: 1.2684x; 1.0115x over previous
import functools

import jax
import jax.numpy as jnp
from jax.experimental import pallas as pl
from jax.experimental.pallas import tpu as pltpu
from jax.experimental.pallas import tpu_sc as plsc

B, S, V, H = 64, 512, 30522, 768
N_TOK = B * S
TWOD = 1024
LN_EPS = 1e-12

NC, NS = 2, 16
NW = NC * NS
NCHUNK = 1
CB = N_TOK // NCHUNK
B_PER_W = CB // NW
CH = 64
NCH = B_PER_W // CH
TB = 512


def _sc_word_gather(weight, ids1d):
    @functools.partial(
        pl.kernel,
        out_type=jax.ShapeDtypeStruct((CB, H), weight.dtype),
        mesh=plsc.VectorSubcoreMesh(
            core_axis_name="c", subcore_axis_name="s", num_cores=NC,
            num_subcores=NS),
        scratch_types=[
            pltpu.VMEM((B_PER_W,), jnp.int32),
            pltpu.VMEM((CH, H), jnp.float32),
            pltpu.VMEM((CH, H), jnp.float32),
            pltpu.SemaphoreType.DMA,
            pltpu.SemaphoreType.DMA,
            pltpu.SemaphoreType.DMA,
            pltpu.SemaphoreType.DMA,
        ],
    )
    def gather_kernel(w_hbm, i_hbm, o_hbm, idx_v, rows0, rows1,
                      gsem0, gsem1, osem0, osem1):
        wid = jax.lax.axis_index("s") * NC + jax.lax.axis_index("c")
        base = wid * B_PER_W
        pltpu.sync_copy(i_hbm.at[pl.ds(base, B_PER_W)], idx_v)

        def g_src(i):
            return w_hbm.at[idx_v.at[pl.ds(i * CH, CH)]]

        def o_dst(i):
            return o_hbm.at[pl.ds(base + i * CH, CH)]

        pltpu.async_copy(g_src(0), rows0, gsem0)
        pltpu.async_copy(g_src(1), rows1, gsem1)

        @pl.loop(0, NCH, step=2)
        def _(i):
            pltpu.make_async_copy(g_src(i), rows0, gsem0).wait()
            pltpu.async_copy(rows0, o_dst(i), osem0)
            pltpu.make_async_copy(g_src(i + 1), rows1, gsem1).wait()
            pltpu.async_copy(rows1, o_dst(i + 1), osem1)
            pltpu.make_async_copy(rows0, o_dst(i), osem0).wait()

            @pl.when(i + 2 < NCH)
            def _():
                pltpu.async_copy(g_src(i + 2), rows0, gsem0)

            pltpu.make_async_copy(rows1, o_dst(i + 1), osem1).wait()

            @pl.when(i + 3 < NCH)
            def _():
                pltpu.async_copy(g_src(i + 3), rows1, gsem1)

    return gather_kernel(weight, ids1d)


NSPLIT = 2
SB = TB // NSPLIT


def _tc_body(bbox_ref, g_ref, tab_ref, p_ref, t_ref, gam_ref, bet_ref, o_ref):
    for k in range(NSPLIT):
        sl = pl.ds(k * SB, SB)
        bb = bbox_ref[sl, :]
        left = bb[:, 0:1]
        upper = bb[:, 1:2]
        right = bb[:, 2:3]
        lower = bb[:, 3:4]
        iota = jax.lax.broadcasted_iota(jnp.int32, (SB, TWOD), 1)

        def onehot(idx):
            return (iota == idx).astype(jnp.bfloat16)

        cc = jnp.concatenate([
            onehot(left) + onehot(right),
            onehot(upper) + onehot(lower),
            onehot(lower - upper),
            onehot(right - left),
        ], axis=1)

        acc = g_ref[sl, :] + p_ref[sl, :] + t_ref[0:1, :]
        acc = acc + jnp.dot(cc, tab_ref[...],
                            preferred_element_type=jnp.float32)

        mean = jnp.mean(acc, axis=1, keepdims=True)
        cen = acc - mean
        var = jnp.mean(cen * cen, axis=1, keepdims=True)
        o_ref[sl, :] = ((cen * jax.lax.rsqrt(var + LN_EPS)) * gam_ref[...]
                        + bet_ref[...])


def _tc_combine(bbox2, gathered, tab4, pe, tte, gam2, bet2):
    return pl.pallas_call(
        _tc_body,
        grid=(CB // TB,),
        in_specs=[
            pl.BlockSpec((TB, 4), lambda i: (i, 0)),
            pl.BlockSpec((TB, H), lambda i: (i, 0)),
            pl.BlockSpec((4 * TWOD, H), lambda i: (0, 0)),
            pl.BlockSpec((S, H), lambda i: (0, 0)),
            pl.BlockSpec((2, H), lambda i: (0, 0)),
            pl.BlockSpec((1, H), lambda i: (0, 0)),
            pl.BlockSpec((1, H), lambda i: (0, 0)),
        ],
        out_specs=pl.BlockSpec((TB, H), lambda i: (i, 0)),
        out_shape=jax.ShapeDtypeStruct((CB, H), jnp.float32),
        compiler_params=pltpu.CompilerParams(
            dimension_semantics=("arbitrary",)),
    )(bbox2, gathered, tab4, pe, tte, gam2, bet2)


def kernel(input_ids, bbox, weight, token_type_embeddings,
           position_embeddings, x_position_embeddings, y_position_embeddings,
           h_position_embeddings, w_position_embeddings, ln_gamma, ln_beta):
    ids1d = input_ids.reshape(N_TOK)
    bbox2 = bbox.reshape(N_TOK, 4)
    tab4 = jnp.concatenate([
        x_position_embeddings, y_position_embeddings,
        h_position_embeddings, w_position_embeddings,
    ], axis=0).astype(jnp.bfloat16)
    gam2 = ln_gamma.reshape(1, H)
    bet2 = ln_beta.reshape(1, H)

    gs = [_sc_word_gather(weight, ids1d[c * CB:(c + 1) * CB])
          for c in range(NCHUNK)]
    outs = []
    for c in range(NCHUNK):
        outs.append(_tc_combine(
            bbox2[c * CB:(c + 1) * CB], gs[c], tab4,
            position_embeddings, token_type_embeddings, gam2, bet2))
    return jnp.concatenate(outs, axis=0).reshape(B, S, H)


# speedup vs baseline: 1.3140x; 1.0360x over previous
import functools

import jax
import jax.numpy as jnp
from jax.experimental import pallas as pl
from jax.experimental.pallas import tpu as pltpu
from jax.experimental.pallas import tpu_sc as plsc

B, S, V, H = 64, 512, 30522, 768
N_TOK = B * S
TWOD = 1024
LN_EPS = 1e-12

NC, NS = 2, 16
NW = NC * NS
NCHUNK = 1
CB = N_TOK // NCHUNK
B_PER_W = CB // NW
CH = 64
NCH = B_PER_W // CH
TB = 1024


def _sc_word_gather(weight, ids1d):
    @functools.partial(
        pl.kernel,
        out_type=jax.ShapeDtypeStruct((CB, H), weight.dtype),
        mesh=plsc.VectorSubcoreMesh(
            core_axis_name="c", subcore_axis_name="s", num_cores=NC,
            num_subcores=NS),
        scratch_types=[
            pltpu.VMEM((B_PER_W,), jnp.int32),
            pltpu.VMEM((CH, H), jnp.float32),
            pltpu.VMEM((CH, H), jnp.float32),
            pltpu.SemaphoreType.DMA,
            pltpu.SemaphoreType.DMA,
            pltpu.SemaphoreType.DMA,
            pltpu.SemaphoreType.DMA,
        ],
    )
    def gather_kernel(w_hbm, i_hbm, o_hbm, idx_v, rows0, rows1,
                      gsem0, gsem1, osem0, osem1):
        wid = jax.lax.axis_index("s") * NC + jax.lax.axis_index("c")
        base = wid * B_PER_W
        pltpu.sync_copy(i_hbm.at[pl.ds(base, B_PER_W)], idx_v)

        def g_src(i):
            return w_hbm.at[idx_v.at[pl.ds(i * CH, CH)]]

        def o_dst(i):
            return o_hbm.at[pl.ds(base + i * CH, CH)]

        pltpu.async_copy(g_src(0), rows0, gsem0)
        pltpu.async_copy(g_src(1), rows1, gsem1)

        @pl.loop(0, NCH, step=2)
        def _(i):
            pltpu.make_async_copy(g_src(i), rows0, gsem0).wait()
            pltpu.async_copy(rows0, o_dst(i), osem0)
            pltpu.make_async_copy(g_src(i + 1), rows1, gsem1).wait()
            pltpu.async_copy(rows1, o_dst(i + 1), osem1)
            pltpu.make_async_copy(rows0, o_dst(i), osem0).wait()

            @pl.when(i + 2 < NCH)
            def _():
                pltpu.async_copy(g_src(i + 2), rows0, gsem0)

            pltpu.make_async_copy(rows1, o_dst(i + 1), osem1).wait()

            @pl.when(i + 3 < NCH)
            def _():
                pltpu.async_copy(g_src(i + 3), rows1, gsem1)

    return gather_kernel(weight, ids1d)


NSPLIT = 4
SB = TB // NSPLIT


def _tc_body(bbox_ref, g_ref, tab_ref, p_ref, t_ref, gam_ref, bet_ref, o_ref):
    for k in range(NSPLIT):
        sl = pl.ds(k * SB, SB)
        bb = bbox_ref[sl, :]
        left = bb[:, 0:1]
        upper = bb[:, 1:2]
        right = bb[:, 2:3]
        lower = bb[:, 3:4]
        iota = jax.lax.broadcasted_iota(jnp.int32, (SB, TWOD), 1)

        def onehot(idx):
            return (iota == idx).astype(jnp.bfloat16)

        cc = jnp.concatenate([
            onehot(left) + onehot(right),
            onehot(upper) + onehot(lower),
            onehot(lower - upper),
            onehot(right - left),
        ], axis=1)

        psl = pl.ds((k * SB) % S, SB)
        acc = g_ref[sl, :] + p_ref[psl, :] + t_ref[0:1, :]
        acc = acc + jnp.dot(cc, tab_ref[...],
                            preferred_element_type=jnp.float32)

        mean = jnp.mean(acc, axis=1, keepdims=True)
        cen = acc - mean
        var = jnp.mean(cen * cen, axis=1, keepdims=True)
        o_ref[sl, :] = ((cen * jax.lax.rsqrt(var + LN_EPS)) * gam_ref[...]
                        + bet_ref[...])


def _tc_combine(bbox2, gathered, tab4, pe, tte, gam2, bet2):
    return pl.pallas_call(
        _tc_body,
        grid=(CB // TB,),
        in_specs=[
            pl.BlockSpec((TB, 4), lambda i: (i, 0)),
            pl.BlockSpec((TB, H), lambda i: (i, 0)),
            pl.BlockSpec((4 * TWOD, H), lambda i: (0, 0)),
            pl.BlockSpec((S, H), lambda i: (0, 0)),
            pl.BlockSpec((2, H), lambda i: (0, 0)),
            pl.BlockSpec((1, H), lambda i: (0, 0)),
            pl.BlockSpec((1, H), lambda i: (0, 0)),
        ],
        out_specs=pl.BlockSpec((TB, H), lambda i: (i, 0)),
        out_shape=jax.ShapeDtypeStruct((CB, H), jnp.float32),
        compiler_params=pltpu.CompilerParams(
            dimension_semantics=("arbitrary",)),
    )(bbox2, gathered, tab4, pe, tte, gam2, bet2)


def kernel(input_ids, bbox, weight, token_type_embeddings,
           position_embeddings, x_position_embeddings, y_position_embeddings,
           h_position_embeddings, w_position_embeddings, ln_gamma, ln_beta):
    ids1d = input_ids.reshape(N_TOK)
    bbox2 = bbox.reshape(N_TOK, 4)
    tab4 = jnp.concatenate([
        x_position_embeddings, y_position_embeddings,
        h_position_embeddings, w_position_embeddings,
    ], axis=0).astype(jnp.bfloat16)
    gam2 = ln_gamma.reshape(1, H)
    bet2 = ln_beta.reshape(1, H)

    gs = [_sc_word_gather(weight, ids1d[c * CB:(c + 1) * CB])
          for c in range(NCHUNK)]
    outs = []
    for c in range(NCHUNK):
        outs.append(_tc_combine(
            bbox2[c * CB:(c + 1) * CB], gs[c], tab4,
            position_embeddings, token_type_embeddings, gam2, bet2))
    return jnp.concatenate(outs, axis=0).reshape(B, S, H)
